# baseline (speedup 1.0000x reference)
import jax
import jax.numpy as jnp
from jax.experimental import pallas as pl
from jax.experimental.pallas import tpu as pltpu

N, DEG, D_IN, D_OUT = 10000, 32, 128, 128
BN = 1000
HD = DEG // 2


def _grn_block(neigh0_ref, neigh1_ref, attn_ref, w_ref, b_ref, out_ref):
    attn = attn_ref[...]
    agg = (
        jnp.sum(neigh0_ref[...] * attn[:, :HD, None], axis=1)
        + jnp.sum(neigh1_ref[...] * attn[:, HD:, None], axis=1)
    )
    proj = jax.lax.dot_general(
        agg, w_ref[...],
        dimension_numbers=(((1,), (1,)), ((), ())),
        preferred_element_type=jnp.float32,
    )
    x = proj + b_ref[...][None, :]
    out_ref[...] = jnp.where(x > 0, x, jnp.exp(x) - 1.0)


def kernel(nodes, neighbors, attention_scores, W, b):
    del nodes
    return pl.pallas_call(
        _grn_block,
        grid=(N // BN,),
        in_specs=[
            pl.BlockSpec((BN, HD, D_IN), lambda i: (i, 0, 0)),
            pl.BlockSpec((BN, HD, D_IN), lambda i: (i, 1, 0)),
            pl.BlockSpec((BN, DEG), lambda i: (i, 0)),
            pl.BlockSpec((D_OUT, D_IN), lambda i: (0, 0)),
            pl.BlockSpec((D_OUT,), lambda i: (0,)),
        ],
        out_specs=pl.BlockSpec((BN, D_OUT), lambda i: (i, 0)),
        out_shape=jax.ShapeDtypeStruct((N, D_OUT), jnp.float32),
        compiler_params=pltpu.CompilerParams(
            dimension_semantics=("arbitrary",),
        ),
    )(neighbors, neighbors, attention_scores, W, b)
